# Initial kernel scaffold; baseline (speedup 1.0000x reference)
import jax
import jax.numpy as jnp
from jax.experimental import pallas as pl


def kernel(x, edges, W_in, b_in, Ws0, bs0, Wn0, Ws1, bs1, Wn1):
    raise NotImplementedError("write your pallas kernel here")


# speedup vs baseline: 4.2704x; 4.2704x over previous
import functools

import jax
import jax.numpy as jnp
from jax import lax
from jax.experimental import pallas as pl
from jax.experimental.pallas import tpu as pltpu
from jax.experimental.pallas import tpu_sc as plsc

N_NODES = 10000
N_EDGES = 320000
IN_DIM = 128
HID = 64

NP = 10240
NC, NS = 2, 16
NW = NC * NS
CHUNK = 128
CW = 80
E_PAD = NW * CW * CHUNK
DEGW = 16
ROWS_PT = NP // NS


def _sc_agg_body(with_deg, h_hbm, srcs_hbm, dsts_hbm, z64_hbm, z16_hbm,
                 ones_hbm, agg_out, deg_out, src_v, dst_v, rows_v, ones_v,
                 agg_sh, deg_sh, sem):
    cid = lax.axis_index("c")
    sid = lax.axis_index("s")
    r0 = sid * ROWS_PT
    pltpu.sync_copy(z64_hbm.at[pl.ds(r0, ROWS_PT)], agg_sh.at[pl.ds(r0, ROWS_PT)])
    if with_deg:
        pltpu.sync_copy(z16_hbm.at[pl.ds(r0, ROWS_PT)], deg_sh.at[pl.ds(r0, ROWS_PT)])
        pltpu.sync_copy(ones_hbm, ones_v)
    wid = cid * NS + sid
    pltpu.sync_copy(srcs_hbm.at[wid], src_v)
    pltpu.sync_copy(dsts_hbm.at[wid], dst_v)
    plsc.subcore_barrier()

    def body(j, carry):
        pltpu.async_copy(h_hbm.at[src_v.at[j]], rows_v, sem).wait()
        pltpu.sync_copy(rows_v, agg_sh.at[dst_v.at[j]], add=True)
        if with_deg:
            pltpu.sync_copy(ones_v, deg_sh.at[dst_v.at[j]], add=True)
        return carry

    lax.fori_loop(0, CW, body, 0)
    plsc.subcore_barrier()
    pltpu.sync_copy(agg_sh.at[pl.ds(r0, ROWS_PT)],
                    agg_out.at[cid, pl.ds(r0, ROWS_PT)])
    if with_deg:
        pltpu.sync_copy(deg_sh.at[pl.ds(r0, ROWS_PT)],
                        deg_out.at[cid, pl.ds(r0, ROWS_PT)])


def _make_sc_agg(with_deg):
    out_type = [jax.ShapeDtypeStruct((NC, NP, HID), jnp.float32)]
    scratch = [
        pltpu.VMEM((CW, CHUNK), jnp.int32),
        pltpu.VMEM((CW, CHUNK), jnp.int32),
        pltpu.VMEM((CHUNK, HID), jnp.float32),
        pltpu.VMEM((CHUNK, DEGW), jnp.float32),
        pltpu.VMEM_SHARED((NP, HID), jnp.float32),
        pltpu.VMEM_SHARED((NP, DEGW), jnp.float32),
        pltpu.SemaphoreType.DMA,
    ]
    if with_deg:
        out_type = out_type + [jax.ShapeDtypeStruct((NC, NP, DEGW), jnp.float32)]

        def fn(h, srcs, dsts, z64, z16, ones, agg_out, deg_out, *scr):
            _sc_agg_body(True, h, srcs, dsts, z64, z16, ones, agg_out,
                         deg_out, *scr)
    else:

        def fn(h, srcs, dsts, z64, z16, ones, agg_out, *scr):
            _sc_agg_body(False, h, srcs, dsts, z64, z16, ones, agg_out,
                         None, *scr)

    return pl.kernel(
        fn,
        out_type=out_type,
        mesh=plsc.VectorSubcoreMesh(core_axis_name="c", subcore_axis_name="s",
                                    num_cores=NC, num_subcores=NS),
        scratch_types=scratch,
        compiler_params=pltpu.CompilerParams(use_tc_tiling_on_sc=False),
    )


_get_sc_agg = functools.cache(_make_sc_agg)

BP = 1024


def _tc_pre_body(x_ref, w_ref, b_ref, o_ref):
    o_ref[...] = jnp.maximum(
        jnp.dot(x_ref[...], w_ref[...], preferred_element_type=jnp.float32)
        + b_ref[...], 0.0)


@jax.jit
def _tc_pre(x, w, b):
    return pl.pallas_call(
        _tc_pre_body,
        grid=(NP // BP,),
        in_specs=[
            pl.BlockSpec((BP, IN_DIM), lambda i: (i, 0)),
            pl.BlockSpec((IN_DIM, HID), lambda i: (0, 0)),
            pl.BlockSpec((1, HID), lambda i: (0, 0)),
        ],
        out_specs=pl.BlockSpec((BP, HID), lambda i: (i, 0)),
        out_shape=jax.ShapeDtypeStruct((NP, HID), jnp.float32),
    )(x, w, b)


def _tc_layer_body(h_ref, agg_ref, deg_ref, ws_ref, bs_ref, wn_ref, o_ref):
    h = h_ref[...]
    agg = agg_ref[0] + agg_ref[1]
    degm = deg_ref[0] + deg_ref[1]
    deg = degm[:, 0:1]
    neigh = agg / jnp.maximum(deg, 1.0)
    o_ref[...] = jnp.maximum(
        jnp.dot(h, ws_ref[...], preferred_element_type=jnp.float32)
        + bs_ref[...]
        + jnp.dot(neigh, wn_ref[...], preferred_element_type=jnp.float32),
        0.0)


@jax.jit
def _tc_layer(h, agg, degm, ws, bs, wn):
    return pl.pallas_call(
        _tc_layer_body,
        grid=(NP // BP,),
        in_specs=[
            pl.BlockSpec((BP, HID), lambda i: (i, 0)),
            pl.BlockSpec((NC, BP, HID), lambda i: (0, i, 0)),
            pl.BlockSpec((NC, BP, DEGW), lambda i: (0, i, 0)),
            pl.BlockSpec((HID, HID), lambda i: (0, 0)),
            pl.BlockSpec((1, HID), lambda i: (0, 0)),
            pl.BlockSpec((HID, HID), lambda i: (0, 0)),
        ],
        out_specs=pl.BlockSpec((BP, HID), lambda i: (i, 0)),
        out_shape=jax.ShapeDtypeStruct((NP, HID), jnp.float32),
    )(h, agg, degm, ws, bs, wn)


def kernel(x, edges, W_in, b_in, Ws0, bs0, Wn0, Ws1, bs1, Wn1):
    src = edges[0].astype(jnp.int32)
    dst = edges[1].astype(jnp.int32)
    pad = E_PAD - N_EDGES
    src = jnp.concatenate([src, jnp.zeros((pad,), jnp.int32)])
    dst = jnp.concatenate([dst, jnp.full((pad,), N_NODES, jnp.int32)])
    srcs = src.reshape(NW, CW, CHUNK)
    dsts = dst.reshape(NW, CW, CHUNK)
    x_p = jnp.pad(x, ((0, NP - N_NODES), (0, 0)))
    z64 = jnp.zeros((NP, HID), jnp.float32)
    z16 = jnp.zeros((NP, DEGW), jnp.float32)
    ones = jnp.ones((CHUNK, DEGW), jnp.float32)

    h0 = _tc_pre(x_p, W_in, b_in.reshape(1, HID))
    agg0, degm = _get_sc_agg(True)(h0, srcs, dsts, z64, z16, ones)
    h1 = _tc_layer(h0, agg0, degm, Ws0, bs0.reshape(1, HID), Wn0)
    (agg1,) = _get_sc_agg(False)(h1, srcs, dsts, z64, z16, ones)
    h2 = _tc_layer(h1, agg1, degm, Ws1, bs1.reshape(1, HID), Wn1)
    return h2[:N_NODES]


# speedup vs baseline: 4.9935x; 1.1693x over previous
import functools

import jax
import jax.numpy as jnp
from jax import lax
from jax.experimental import pallas as pl
from jax.experimental.pallas import tpu as pltpu
from jax.experimental.pallas import tpu_sc as plsc

N_NODES = 10000
N_EDGES = 320000
IN_DIM = 128
HID = 64

NP = 10240
NC, NS = 2, 16
NW = NC * NS
CHUNK = 128
CW = 80
E_PAD = NW * CW * CHUNK
DEGW = 16
ROWS_PT = NP // NS


def _sc_agg_body(with_deg, h_hbm, srcs_hbm, dsts_hbm, z64_hbm, z16_hbm,
                 ones_hbm, agg_out, deg_out, src_v, dst_v, rows0, rows1,
                 ones_v, agg_sh, deg_sh, sem0, sem1):
    cid = lax.axis_index("c")
    sid = lax.axis_index("s")
    r0 = sid * ROWS_PT
    pltpu.sync_copy(z64_hbm.at[pl.ds(r0, ROWS_PT)], agg_sh.at[pl.ds(r0, ROWS_PT)])
    if with_deg:
        pltpu.sync_copy(z16_hbm.at[pl.ds(r0, ROWS_PT)], deg_sh.at[pl.ds(r0, ROWS_PT)])
        pltpu.sync_copy(ones_hbm, ones_v)
    wid = cid * NS + sid
    pltpu.sync_copy(srcs_hbm.at[wid], src_v)
    pltpu.sync_copy(dsts_hbm.at[wid], dst_v)
    plsc.subcore_barrier()

    def start_g(jj, buf, sem):
        pltpu.async_copy(h_hbm.at[src_v.at[jj]], buf, sem)

    def wait_g(jj, buf, sem):
        pltpu.make_async_copy(h_hbm.at[src_v.at[jj]], buf, sem).wait()

    start_g(0, rows0, sem0)
    start_g(1, rows1, sem1)

    def body(i, carry):
        j = 2 * i
        wait_g(j, rows0, sem0)
        pltpu.sync_copy(rows0, agg_sh.at[dst_v.at[j]], add=True)

        @pl.when(j + 2 < CW)
        def _():
            start_g(j + 2, rows0, sem0)

        wait_g(j + 1, rows1, sem1)
        pltpu.sync_copy(rows1, agg_sh.at[dst_v.at[j + 1]], add=True)

        @pl.when(j + 3 < CW)
        def _():
            start_g(j + 3, rows1, sem1)

        if with_deg:
            pltpu.sync_copy(ones_v, deg_sh.at[dst_v.at[j]], add=True)
            pltpu.sync_copy(ones_v, deg_sh.at[dst_v.at[j + 1]], add=True)
        return carry

    lax.fori_loop(0, CW // 2, body, 0)
    plsc.subcore_barrier()
    pltpu.sync_copy(agg_sh.at[pl.ds(r0, ROWS_PT)],
                    agg_out.at[cid, pl.ds(r0, ROWS_PT)])
    if with_deg:
        pltpu.sync_copy(deg_sh.at[pl.ds(r0, ROWS_PT)],
                        deg_out.at[cid, pl.ds(r0, ROWS_PT)])


def _make_sc_agg(with_deg):
    out_type = [jax.ShapeDtypeStruct((NC, NP, HID), jnp.float32)]
    scratch = [
        pltpu.VMEM((CW, CHUNK), jnp.int32),
        pltpu.VMEM((CW, CHUNK), jnp.int32),
        pltpu.VMEM((CHUNK, HID), jnp.float32),
        pltpu.VMEM((CHUNK, HID), jnp.float32),
        pltpu.VMEM((CHUNK, DEGW), jnp.float32),
        pltpu.VMEM_SHARED((NP, HID), jnp.float32),
        pltpu.VMEM_SHARED((NP, DEGW), jnp.float32),
        pltpu.SemaphoreType.DMA,
        pltpu.SemaphoreType.DMA,
    ]
    if with_deg:
        out_type = out_type + [jax.ShapeDtypeStruct((NC, NP, DEGW), jnp.float32)]

        def fn(h, srcs, dsts, z64, z16, ones, agg_out, deg_out, *scr):
            _sc_agg_body(True, h, srcs, dsts, z64, z16, ones, agg_out,
                         deg_out, *scr)
    else:

        def fn(h, srcs, dsts, z64, z16, ones, agg_out, *scr):
            _sc_agg_body(False, h, srcs, dsts, z64, z16, ones, agg_out,
                         None, *scr)

    return pl.kernel(
        fn,
        out_type=out_type,
        mesh=plsc.VectorSubcoreMesh(core_axis_name="c", subcore_axis_name="s",
                                    num_cores=NC, num_subcores=NS),
        scratch_types=scratch,
        compiler_params=pltpu.CompilerParams(use_tc_tiling_on_sc=False),
    )


_get_sc_agg = functools.cache(_make_sc_agg)

BP = 1024


def _tc_pre_body(x_ref, w_ref, b_ref, o_ref):
    o_ref[...] = jnp.maximum(
        jnp.dot(x_ref[...], w_ref[...], preferred_element_type=jnp.float32)
        + b_ref[...], 0.0)


@jax.jit
def _tc_pre(x, w, b):
    return pl.pallas_call(
        _tc_pre_body,
        grid=(NP // BP,),
        in_specs=[
            pl.BlockSpec((BP, IN_DIM), lambda i: (i, 0)),
            pl.BlockSpec((IN_DIM, HID), lambda i: (0, 0)),
            pl.BlockSpec((1, HID), lambda i: (0, 0)),
        ],
        out_specs=pl.BlockSpec((BP, HID), lambda i: (i, 0)),
        out_shape=jax.ShapeDtypeStruct((NP, HID), jnp.float32),
    )(x, w, b)


def _tc_layer_body(h_ref, agg_ref, deg_ref, ws_ref, bs_ref, wn_ref, o_ref):
    h = h_ref[...]
    agg = agg_ref[0] + agg_ref[1]
    degm = deg_ref[0] + deg_ref[1]
    deg = degm[:, 0:1]
    neigh = agg / jnp.maximum(deg, 1.0)
    o_ref[...] = jnp.maximum(
        jnp.dot(h, ws_ref[...], preferred_element_type=jnp.float32)
        + bs_ref[...]
        + jnp.dot(neigh, wn_ref[...], preferred_element_type=jnp.float32),
        0.0)


@jax.jit
def _tc_layer(h, agg, degm, ws, bs, wn):
    return pl.pallas_call(
        _tc_layer_body,
        grid=(NP // BP,),
        in_specs=[
            pl.BlockSpec((BP, HID), lambda i: (i, 0)),
            pl.BlockSpec((NC, BP, HID), lambda i: (0, i, 0)),
            pl.BlockSpec((NC, BP, DEGW), lambda i: (0, i, 0)),
            pl.BlockSpec((HID, HID), lambda i: (0, 0)),
            pl.BlockSpec((1, HID), lambda i: (0, 0)),
            pl.BlockSpec((HID, HID), lambda i: (0, 0)),
        ],
        out_specs=pl.BlockSpec((BP, HID), lambda i: (i, 0)),
        out_shape=jax.ShapeDtypeStruct((NP, HID), jnp.float32),
    )(h, agg, degm, ws, bs, wn)


def kernel(x, edges, W_in, b_in, Ws0, bs0, Wn0, Ws1, bs1, Wn1):
    src = edges[0].astype(jnp.int32)
    dst = edges[1].astype(jnp.int32)
    pad = E_PAD - N_EDGES
    src = jnp.concatenate([src, jnp.zeros((pad,), jnp.int32)])
    dst = jnp.concatenate([dst, jnp.full((pad,), N_NODES, jnp.int32)])
    srcs = src.reshape(NW, CW, CHUNK)
    dsts = dst.reshape(NW, CW, CHUNK)
    x_p = jnp.pad(x, ((0, NP - N_NODES), (0, 0)))
    z64 = jnp.zeros((NP, HID), jnp.float32)
    z16 = jnp.zeros((NP, DEGW), jnp.float32)
    ones = jnp.ones((CHUNK, DEGW), jnp.float32)

    h0 = _tc_pre(x_p, W_in, b_in.reshape(1, HID))
    agg0, degm = _get_sc_agg(True)(h0, srcs, dsts, z64, z16, ones)
    h1 = _tc_layer(h0, agg0, degm, Ws0, bs0.reshape(1, HID), Wn0)
    (agg1,) = _get_sc_agg(False)(h1, srcs, dsts, z64, z16, ones)
    h2 = _tc_layer(h1, agg1, degm, Ws1, bs1.reshape(1, HID), Wn1)
    return h2[:N_NODES]


# speedup vs baseline: 9.6596x; 1.9345x over previous
import functools

import jax
import jax.numpy as jnp
from jax import lax
from jax.experimental import pallas as pl
from jax.experimental.pallas import tpu as pltpu
from jax.experimental.pallas import tpu_sc as plsc

N_NODES = 10000
N_EDGES = 320000
IN_DIM = 128
HID = 64

NP = 10240
NC, NS = 2, 16
NW = NC * NS
CHUNK = 128
CW = 80
E_PAD = NW * CW * CHUNK
DEGW = 16
ROWS_PT = NP // NS


def _sc_agg_body(with_deg, h_hbm, srcs_hbm, dsts_hbm, z64_hbm, z16_hbm,
                 ones_hbm, agg_out, deg_out, src_v, dst_v, rows0, rows1,
                 ones_v, agg_sh, deg_sh, h_sh, sem0, sem1):
    cid = lax.axis_index("c")
    sid = lax.axis_index("s")
    r0 = sid * ROWS_PT
    pltpu.sync_copy(h_hbm.at[pl.ds(r0, ROWS_PT)], h_sh.at[pl.ds(r0, ROWS_PT)])
    pltpu.sync_copy(z64_hbm.at[pl.ds(r0, ROWS_PT)], agg_sh.at[pl.ds(r0, ROWS_PT)])
    if with_deg:
        pltpu.sync_copy(z16_hbm.at[pl.ds(r0, ROWS_PT)], deg_sh.at[pl.ds(r0, ROWS_PT)])
        pltpu.sync_copy(ones_hbm, ones_v)
    wid = cid * NS + sid
    pltpu.sync_copy(srcs_hbm.at[wid], src_v)
    pltpu.sync_copy(dsts_hbm.at[wid], dst_v)
    plsc.subcore_barrier()

    def start_g(jj, buf, sem):
        pltpu.async_copy(h_sh.at[src_v.at[jj]], buf, sem)

    def wait_g(jj, buf, sem):
        pltpu.make_async_copy(h_sh.at[src_v.at[jj]], buf, sem).wait()

    start_g(0, rows0, sem0)
    start_g(1, rows1, sem1)

    def body(i, carry):
        j = 2 * i
        wait_g(j, rows0, sem0)
        pltpu.sync_copy(rows0, agg_sh.at[dst_v.at[j]], add=True)

        @pl.when(j + 2 < CW)
        def _():
            start_g(j + 2, rows0, sem0)

        wait_g(j + 1, rows1, sem1)
        pltpu.sync_copy(rows1, agg_sh.at[dst_v.at[j + 1]], add=True)

        @pl.when(j + 3 < CW)
        def _():
            start_g(j + 3, rows1, sem1)

        if with_deg:
            pltpu.sync_copy(ones_v, deg_sh.at[dst_v.at[j]], add=True)
            pltpu.sync_copy(ones_v, deg_sh.at[dst_v.at[j + 1]], add=True)
        return carry

    lax.fori_loop(0, CW // 2, body, 0)
    plsc.subcore_barrier()
    pltpu.sync_copy(agg_sh.at[pl.ds(r0, ROWS_PT)],
                    agg_out.at[cid, pl.ds(r0, ROWS_PT)])
    if with_deg:
        pltpu.sync_copy(deg_sh.at[pl.ds(r0, ROWS_PT)],
                        deg_out.at[cid, pl.ds(r0, ROWS_PT)])


def _make_sc_agg(with_deg):
    out_type = [jax.ShapeDtypeStruct((NC, NP, HID), jnp.float32)]
    scratch = [
        pltpu.VMEM((CW, CHUNK), jnp.int32),
        pltpu.VMEM((CW, CHUNK), jnp.int32),
        pltpu.VMEM((CHUNK, HID), jnp.float32),
        pltpu.VMEM((CHUNK, HID), jnp.float32),
        pltpu.VMEM((CHUNK, DEGW), jnp.float32),
        pltpu.VMEM_SHARED((NP, HID), jnp.float32),
        pltpu.VMEM_SHARED((NP, DEGW), jnp.float32),
        pltpu.VMEM_SHARED((NP, HID), jnp.float32),
        pltpu.SemaphoreType.DMA,
        pltpu.SemaphoreType.DMA,
    ]
    if with_deg:
        out_type = out_type + [jax.ShapeDtypeStruct((NC, NP, DEGW), jnp.float32)]

        def fn(h, srcs, dsts, z64, z16, ones, agg_out, deg_out, *scr):
            _sc_agg_body(True, h, srcs, dsts, z64, z16, ones, agg_out,
                         deg_out, *scr)
    else:

        def fn(h, srcs, dsts, z64, z16, ones, agg_out, *scr):
            _sc_agg_body(False, h, srcs, dsts, z64, z16, ones, agg_out,
                         None, *scr)

    return pl.kernel(
        fn,
        out_type=out_type,
        mesh=plsc.VectorSubcoreMesh(core_axis_name="c", subcore_axis_name="s",
                                    num_cores=NC, num_subcores=NS),
        scratch_types=scratch,
        compiler_params=pltpu.CompilerParams(use_tc_tiling_on_sc=False),
    )


_get_sc_agg = functools.cache(_make_sc_agg)

BP = 1024


def _tc_pre_body(x_ref, w_ref, b_ref, o_ref):
    o_ref[...] = jnp.maximum(
        jnp.dot(x_ref[...], w_ref[...], preferred_element_type=jnp.float32)
        + b_ref[...], 0.0)


@jax.jit
def _tc_pre(x, w, b):
    return pl.pallas_call(
        _tc_pre_body,
        grid=(NP // BP,),
        in_specs=[
            pl.BlockSpec((BP, IN_DIM), lambda i: (i, 0)),
            pl.BlockSpec((IN_DIM, HID), lambda i: (0, 0)),
            pl.BlockSpec((1, HID), lambda i: (0, 0)),
        ],
        out_specs=pl.BlockSpec((BP, HID), lambda i: (i, 0)),
        out_shape=jax.ShapeDtypeStruct((NP, HID), jnp.float32),
    )(x, w, b)


def _tc_layer_body(h_ref, agg_ref, deg_ref, ws_ref, bs_ref, wn_ref, o_ref):
    h = h_ref[...]
    agg = agg_ref[0] + agg_ref[1]
    degm = deg_ref[0] + deg_ref[1]
    deg = degm[:, 0:1]
    neigh = agg / jnp.maximum(deg, 1.0)
    o_ref[...] = jnp.maximum(
        jnp.dot(h, ws_ref[...], preferred_element_type=jnp.float32)
        + bs_ref[...]
        + jnp.dot(neigh, wn_ref[...], preferred_element_type=jnp.float32),
        0.0)


@jax.jit
def _tc_layer(h, agg, degm, ws, bs, wn):
    return pl.pallas_call(
        _tc_layer_body,
        grid=(NP // BP,),
        in_specs=[
            pl.BlockSpec((BP, HID), lambda i: (i, 0)),
            pl.BlockSpec((NC, BP, HID), lambda i: (0, i, 0)),
            pl.BlockSpec((NC, BP, DEGW), lambda i: (0, i, 0)),
            pl.BlockSpec((HID, HID), lambda i: (0, 0)),
            pl.BlockSpec((1, HID), lambda i: (0, 0)),
            pl.BlockSpec((HID, HID), lambda i: (0, 0)),
        ],
        out_specs=pl.BlockSpec((BP, HID), lambda i: (i, 0)),
        out_shape=jax.ShapeDtypeStruct((NP, HID), jnp.float32),
    )(h, agg, degm, ws, bs, wn)


def kernel(x, edges, W_in, b_in, Ws0, bs0, Wn0, Ws1, bs1, Wn1):
    src = edges[0].astype(jnp.int32)
    dst = edges[1].astype(jnp.int32)
    pad = E_PAD - N_EDGES
    src = jnp.concatenate([src, jnp.zeros((pad,), jnp.int32)])
    dst = jnp.concatenate([dst, jnp.full((pad,), N_NODES, jnp.int32)])
    srcs = src.reshape(NW, CW, CHUNK)
    dsts = dst.reshape(NW, CW, CHUNK)
    x_p = jnp.pad(x, ((0, NP - N_NODES), (0, 0)))
    z64 = jnp.zeros((NP, HID), jnp.float32)
    z16 = jnp.zeros((NP, DEGW), jnp.float32)
    ones = jnp.ones((CHUNK, DEGW), jnp.float32)

    h0 = _tc_pre(x_p, W_in, b_in.reshape(1, HID))
    agg0, degm = _get_sc_agg(True)(h0, srcs, dsts, z64, z16, ones)
    h1 = _tc_layer(h0, agg0, degm, Ws0, bs0.reshape(1, HID), Wn0)
    (agg1,) = _get_sc_agg(False)(h1, srcs, dsts, z64, z16, ones)
    h2 = _tc_layer(h1, agg1, degm, Ws1, bs1.reshape(1, HID), Wn1)
    return h2[:N_NODES]


# speedup vs baseline: 10.0332x; 1.0387x over previous
import functools

import jax
import jax.numpy as jnp
from jax import lax
from jax.experimental import pallas as pl
from jax.experimental.pallas import tpu as pltpu
from jax.experimental.pallas import tpu_sc as plsc

N_NODES = 10000
N_EDGES = 320000
IN_DIM = 128
HID = 64

NC, NS = 2, 16
NW = NC * NS
CHUNK = 125
CW = 80
DEGW = 16
ROWS_PT = N_NODES // NS


def _sc_agg_body(with_deg, h_hbm, edges_hbm, z64_hbm, z16_hbm,
                 ones_hbm, agg_out, deg_out, src_v, dst_v, rows0, rows1,
                 ones_v, agg_sh, deg_sh, h_sh, sem0, sem1):
    cid = lax.axis_index("c")
    sid = lax.axis_index("s")
    r0 = sid * ROWS_PT
    pltpu.sync_copy(h_hbm.at[pl.ds(r0, ROWS_PT)], h_sh.at[pl.ds(r0, ROWS_PT)])
    pltpu.sync_copy(z64_hbm.at[pl.ds(r0, ROWS_PT)], agg_sh.at[pl.ds(r0, ROWS_PT)])
    if with_deg:
        pltpu.sync_copy(z16_hbm.at[pl.ds(r0, ROWS_PT)], deg_sh.at[pl.ds(r0, ROWS_PT)])
        pltpu.sync_copy(ones_hbm, ones_v)
    wid = cid * NS + sid
    pltpu.sync_copy(edges_hbm.at[0, wid], src_v)
    pltpu.sync_copy(edges_hbm.at[1, wid], dst_v)
    plsc.subcore_barrier()

    def start_g(jj, buf, sem):
        pltpu.async_copy(h_sh.at[src_v.at[jj]], buf, sem)

    def wait_g(jj, buf, sem):
        pltpu.make_async_copy(h_sh.at[src_v.at[jj]], buf, sem).wait()

    start_g(0, rows0, sem0)
    start_g(1, rows1, sem1)

    def body(i, carry):
        j = 2 * i
        wait_g(j, rows0, sem0)
        pltpu.sync_copy(rows0, agg_sh.at[dst_v.at[j]], add=True)

        @pl.when(j + 2 < CW)
        def _():
            start_g(j + 2, rows0, sem0)

        wait_g(j + 1, rows1, sem1)
        pltpu.sync_copy(rows1, agg_sh.at[dst_v.at[j + 1]], add=True)

        @pl.when(j + 3 < CW)
        def _():
            start_g(j + 3, rows1, sem1)

        if with_deg:
            pltpu.sync_copy(ones_v, deg_sh.at[dst_v.at[j]], add=True)
            pltpu.sync_copy(ones_v, deg_sh.at[dst_v.at[j + 1]], add=True)
        return carry

    lax.fori_loop(0, CW // 2, body, 0)
    plsc.subcore_barrier()
    pltpu.sync_copy(agg_sh.at[pl.ds(r0, ROWS_PT)],
                    agg_out.at[cid, pl.ds(r0, ROWS_PT)])
    if with_deg:
        pltpu.sync_copy(deg_sh.at[pl.ds(r0, ROWS_PT)],
                        deg_out.at[cid, pl.ds(r0, ROWS_PT)])


def _make_sc_agg(with_deg):
    out_type = [jax.ShapeDtypeStruct((NC, N_NODES, HID), jnp.float32)]
    scratch = [
        pltpu.VMEM((CW, CHUNK), jnp.int32),
        pltpu.VMEM((CW, CHUNK), jnp.int32),
        pltpu.VMEM((CHUNK, HID), jnp.float32),
        pltpu.VMEM((CHUNK, HID), jnp.float32),
        pltpu.VMEM((CHUNK, DEGW), jnp.float32),
        pltpu.VMEM_SHARED((N_NODES, HID), jnp.float32),
        pltpu.VMEM_SHARED((N_NODES, DEGW), jnp.float32),
        pltpu.VMEM_SHARED((N_NODES, HID), jnp.float32),
        pltpu.SemaphoreType.DMA,
        pltpu.SemaphoreType.DMA,
    ]
    if with_deg:
        out_type = out_type + [jax.ShapeDtypeStruct((NC, N_NODES, DEGW), jnp.float32)]

        def fn(h, edges, z64, z16, ones, agg_out, deg_out, *scr):
            _sc_agg_body(True, h, edges, z64, z16, ones, agg_out,
                         deg_out, *scr)
    else:

        def fn(h, edges, z64, z16, ones, agg_out, *scr):
            _sc_agg_body(False, h, edges, z64, z16, ones, agg_out,
                         None, *scr)

    return pl.kernel(
        fn,
        out_type=out_type,
        mesh=plsc.VectorSubcoreMesh(core_axis_name="c", subcore_axis_name="s",
                                    num_cores=NC, num_subcores=NS),
        scratch_types=scratch,
        compiler_params=pltpu.CompilerParams(use_tc_tiling_on_sc=False),
    )


_get_sc_agg = functools.cache(_make_sc_agg)

BP = 1000


def _tc_pre_body(x_ref, w_ref, b_ref, o_ref):
    o_ref[...] = jnp.maximum(
        jnp.dot(x_ref[...], w_ref[...], preferred_element_type=jnp.float32)
        + b_ref[...], 0.0)


@jax.jit
def _tc_pre(x, w, b):
    return pl.pallas_call(
        _tc_pre_body,
        grid=(N_NODES // BP,),
        in_specs=[
            pl.BlockSpec((BP, IN_DIM), lambda i: (i, 0)),
            pl.BlockSpec((IN_DIM, HID), lambda i: (0, 0)),
            pl.BlockSpec((1, HID), lambda i: (0, 0)),
        ],
        out_specs=pl.BlockSpec((BP, HID), lambda i: (i, 0)),
        out_shape=jax.ShapeDtypeStruct((N_NODES, HID), jnp.float32),
    )(x, w, b)


def _tc_layer_body(h_ref, agg_ref, deg_ref, ws_ref, bs_ref, wn_ref, o_ref):
    h = h_ref[...]
    agg = agg_ref[0] + agg_ref[1]
    degm = deg_ref[0] + deg_ref[1]
    deg = degm[:, 0:1]
    neigh = agg / jnp.maximum(deg, 1.0)
    o_ref[...] = jnp.maximum(
        jnp.dot(h, ws_ref[...], preferred_element_type=jnp.float32)
        + bs_ref[...]
        + jnp.dot(neigh, wn_ref[...], preferred_element_type=jnp.float32),
        0.0)


@jax.jit
def _tc_layer(h, agg, degm, ws, bs, wn):
    return pl.pallas_call(
        _tc_layer_body,
        grid=(N_NODES // BP,),
        in_specs=[
            pl.BlockSpec((BP, HID), lambda i: (i, 0)),
            pl.BlockSpec((NC, BP, HID), lambda i: (0, i, 0)),
            pl.BlockSpec((NC, BP, DEGW), lambda i: (0, i, 0)),
            pl.BlockSpec((HID, HID), lambda i: (0, 0)),
            pl.BlockSpec((1, HID), lambda i: (0, 0)),
            pl.BlockSpec((HID, HID), lambda i: (0, 0)),
        ],
        out_specs=pl.BlockSpec((BP, HID), lambda i: (i, 0)),
        out_shape=jax.ShapeDtypeStruct((N_NODES, HID), jnp.float32),
    )(h, agg, degm, ws, bs, wn)


def kernel(x, edges, W_in, b_in, Ws0, bs0, Wn0, Ws1, bs1, Wn1):
    ed = edges.astype(jnp.int32).reshape(2, NW, CW, CHUNK)
    z64 = jnp.zeros((N_NODES, HID), jnp.float32)
    z16 = jnp.zeros((N_NODES, DEGW), jnp.float32)
    ones = jnp.ones((CHUNK, DEGW), jnp.float32)

    h0 = _tc_pre(x, W_in, b_in.reshape(1, HID))
    agg0, degm = _get_sc_agg(True)(h0, ed, z64, z16, ones)
    h1 = _tc_layer(h0, agg0, degm, Ws0, bs0.reshape(1, HID), Wn0)
    (agg1,) = _get_sc_agg(False)(h1, ed, z64, z16, ones)
    h2 = _tc_layer(h1, agg1, degm, Ws1, bs1.reshape(1, HID), Wn1)
    return h2
